# baseline (speedup 1.0000x reference)
import dataclasses
import functools

import jax
import jax.numpy as jnp
from jax import lax
from jax.experimental import pallas as pl
from jax.experimental.pallas import tpu as pltpu
from jax.experimental.pallas import tpu_sc as plsc

NN = 10000
N_PAD = 10240
EE = 320000
DD = 128
NC = 2
NS = 16
CHUNK = 128
N_CHUNKS = 80
DEGW = 16
E_PAD = NC * NS * N_CHUNKS * CHUNK
STRIPE = N_PAD // NS
ROW_BLK = 1024
N_BLKS = N_PAD // ROW_BLK


_USE_SC_HIST = True
PACK_WIN = N_CHUNKS // 2


def _sc_params(with_hist):
    cp = pltpu.CompilerParams(use_tc_tiling_on_sc=False)
    if with_hist and "needs_layout_passes" in pltpu.CompilerParams.__dataclass_fields__:
        cp = dataclasses.replace(cp, needs_layout_passes=False)
    return cp


def _make_segsum(with_deg):
    mesh = plsc.VectorSubcoreMesh(core_axis_name="c", subcore_axis_name="s")
    out_type = [jax.ShapeDtypeStruct((NC, N_PAD, DD), jnp.float32)]
    pack_rows = PACK_WIN if with_deg else N_CHUNKS
    scratch = [
        pltpu.VMEM_SHARED((N_PAD, DD), jnp.float32),
        pltpu.VMEM((pack_rows, CHUNK), jnp.int32),
        pltpu.VMEM((CHUNK,), jnp.int32),
        pltpu.VMEM((CHUNK,), jnp.int32),
        pltpu.VMEM((CHUNK,), jnp.int32),
        pltpu.VMEM((CHUNK,), jnp.int32),
        pltpu.VMEM((CHUNK, DD), jnp.float32),
        pltpu.VMEM((CHUNK, DD), jnp.float32),
    ]
    scratch += [pltpu.SemaphoreType.DMA] * 4
    if with_deg:
        out_type.append(
            jax.ShapeDtypeStruct((NC * NS, N_PAD // DD, DD), jnp.float32))
        if _USE_SC_HIST:
            scratch.insert(6, pltpu.VMEM((N_PAD // DD, DD), jnp.float32))

    @functools.partial(
        pl.kernel,
        mesh=mesh,
        compiler_params=_sc_params(with_deg and _USE_SC_HIST),
        out_type=tuple(out_type) if with_deg else out_type[0],
        scratch_types=scratch,
    )
    def seg(table_hbm, packed_hbm, zeros_hbm, *refs):
        refs = list(refs)
        out_hbm = refs.pop(0)
        deg_hbm = refs.pop(0) if with_deg else None
        acc_sh = refs.pop(0)
        packed = refs.pop(0)
        idx_s = [refs.pop(0), refs.pop(0)]
        idx_d = [refs.pop(0), refs.pop(0)]
        hist = refs.pop(0) if (with_deg and _USE_SC_HIST) else None
        rows = [refs.pop(0), refs.pop(0)]
        gsem = [refs.pop(0), refs.pop(0)]
        ssem = [refs.pop(0), refs.pop(0)]

        c = lax.axis_index("c")
        s = lax.axis_index("s")
        wid = c * NS + s
        pltpu.sync_copy(zeros_hbm.at[pl.ds(0, STRIPE)],
                        acc_sh.at[pl.ds(s * STRIPE, STRIPE)])
        if with_deg:
            pltpu.sync_copy(packed_hbm.at[wid].at[pl.ds(0, PACK_WIN)], packed)
        else:
            pltpu.sync_copy(packed_hbm.at[wid], packed)
        if with_deg and _USE_SC_HIST:
            pltpu.sync_copy(zeros_hbm.at[pl.ds(0, N_PAD // DD)], hist)
        plsc.subcore_barrier()

        def unpack(k, b):
            lk = jnp.where(k >= PACK_WIN, k - PACK_WIN, k) if with_deg else k
            for c0 in range(0, CHUNK, 16):
                w = packed[lk, pl.ds(c0, 16)]
                d = lax.shift_right_logical(w, 16)
                idx_s[b][pl.ds(c0, 16)] = lax.bitwise_and(w, 0xFFFF)
                idx_d[b][pl.ds(c0, 16)] = d
                if with_deg and _USE_SC_HIST:
                    plsc.addupdate_scatter(
                        hist,
                        [lax.shift_right_logical(d, 7),
                         lax.bitwise_and(d, 0x7F)],
                        jnp.full((16,), 1.0, jnp.float32))

        def g_start(k, b):
            pltpu.async_copy(table_hbm.at[idx_s[b]], rows[b], gsem[b])

        def g_wait(b):
            pltpu.make_async_copy(table_hbm.at[pl.ds(0, CHUNK)], rows[b],
                                  gsem[b]).wait()

        def s_start(k, b):
            pltpu.async_copy(rows[b], acc_sh.at[idx_d[b]], ssem[b], add=True)

        def s_wait(b):
            pltpu.make_async_copy(table_hbm.at[pl.ds(0, CHUNK)], rows[b],
                                  ssem[b]).wait()

        @pl.loop(0, N_CHUNKS)
        def _(k):
            if with_deg:
                @pl.when(k == PACK_WIN)
                def _():
                    pltpu.sync_copy(
                        packed_hbm.at[wid].at[pl.ds(PACK_WIN, PACK_WIN)],
                        packed)
            unpack(k, 0)
            g_start(k, 0)
            g_wait(0)
            s_start(k, 0)
            s_wait(0)

        if with_deg and _USE_SC_HIST:
            pltpu.sync_copy(hist, deg_hbm.at[wid])
        plsc.subcore_barrier()
        pltpu.sync_copy(acc_sh.at[pl.ds(s * STRIPE, STRIPE)],
                        out_hbm.at[c].at[pl.ds(s * STRIPE, STRIPE)])

    return seg


_segsum_deg = _make_segsum(True)
_segsum_plain = _make_segsum(False)


ROW_SUB = ROW_BLK // DD


def _tc1_body(acc_ref, deg_ref, x_ref, wl_ref, wr_ref, b_ref, h_ref, inv_ref):
    agg = acc_ref[0] + acc_ref[1]
    deg = jnp.sum(deg_ref[...], axis=0)
    inv = 1.0 / jnp.maximum(deg, 1.0)
    m = (agg * inv[:, :, None]).reshape(ROW_BLK, DD)
    h = (jnp.dot(m, wl_ref[...], preferred_element_type=jnp.float32)
         + jnp.dot(x_ref[...], wr_ref[...], preferred_element_type=jnp.float32)
         + b_ref[...])
    h_ref[...] = jnp.maximum(h, 0.0)
    inv_ref[...] = inv


def _tc2_body(acc_ref, h_ref, inv_ref, wl_ref, wr_ref, b_ref, whd_ref, bhd_ref,
              o1_ref, o2_ref):
    ssum = acc_ref[0] + acc_ref[1]
    m = (ssum * inv_ref[...][:, :, None]).reshape(ROW_BLK, DD)
    h2 = (jnp.dot(m, wl_ref[...], preferred_element_type=jnp.float32)
          + jnp.dot(h_ref[...], wr_ref[...], preferred_element_type=jnp.float32)
          + b_ref[...])
    h2 = jnp.maximum(h2, 0.0)
    t = jnp.dot(h2, whd_ref[...], preferred_element_type=jnp.float32) + bhd_ref[...]
    preds = t[:, 0:1]
    diffs = jax.nn.sigmoid(t[:, 1:2])
    o1_ref[...] = jnp.broadcast_to(preds - diffs, (ROW_BLK, DD))
    o2_ref[...] = jnp.broadcast_to(preds + diffs, (ROW_BLK, DD))


def _full(shape):
    return pl.BlockSpec(shape, lambda j: tuple(0 for _ in shape))


def _tc_layer1(acc1, deg, x_pad, W1_l, W1_r, b1):
    return pl.pallas_call(
        _tc1_body,
        grid=(N_BLKS,),
        in_specs=[
            pl.BlockSpec((NC, ROW_SUB, DD, DD), lambda j: (0, j, 0, 0)),
            pl.BlockSpec((NC * NS, ROW_SUB, DD), lambda j: (0, j, 0)),
            pl.BlockSpec((ROW_BLK, DD), lambda j: (j, 0)),
            _full((DD, DD)),
            _full((DD, DD)),
            _full((1, DD)),
        ],
        out_specs=[
            pl.BlockSpec((ROW_BLK, DD), lambda j: (j, 0)),
            pl.BlockSpec((ROW_SUB, DD), lambda j: (j, 0)),
        ],
        out_shape=[
            jax.ShapeDtypeStruct((N_PAD, DD), jnp.float32),
            jax.ShapeDtypeStruct((N_PAD // DD, DD), jnp.float32),
        ],
    )(acc1, deg, x_pad, W1_l, W1_r, b1)


def _tc_layer2(acc2, h, inv, W2_l, W2_r, b2, W_hd, b_hd):
    return pl.pallas_call(
        _tc2_body,
        grid=(N_BLKS,),
        in_specs=[
            pl.BlockSpec((NC, ROW_SUB, DD, DD), lambda j: (0, j, 0, 0)),
            pl.BlockSpec((ROW_BLK, DD), lambda j: (j, 0)),
            pl.BlockSpec((ROW_SUB, DD), lambda j: (j, 0)),
            _full((DD, DD)),
            _full((DD, DD)),
            _full((1, DD)),
            _full((DD, DD)),
            _full((1, DD)),
        ],
        out_specs=[
            pl.BlockSpec((ROW_BLK, DD), lambda j: (j, 0)),
            pl.BlockSpec((ROW_BLK, DD), lambda j: (j, 0)),
        ],
        out_shape=[
            jax.ShapeDtypeStruct((N_PAD, DD), jnp.float32),
            jax.ShapeDtypeStruct((N_PAD, DD), jnp.float32),
        ],
    )(acc2, h, inv, W2_l, W2_r, b2, W_hd, b_hd)


def kernel(x, edge_index, W1_l, W1_r, b1, W2_l, W2_r, b2, W_pred, b_pred,
           W_diff, b_diff):
    f32 = jnp.float32
    packed = jnp.bitwise_or(edge_index[0],
                            jnp.left_shift(edge_index[1], 16))
    packed = jnp.concatenate(
        [packed, jnp.full((E_PAD - EE,), NN << 16, jnp.int32)]).reshape(
            NC * NS, N_CHUNKS, CHUNK)
    x_pad = jnp.concatenate([x, jnp.zeros((N_PAD - NN, DD), f32)], axis=0)
    zeros_d = jnp.zeros((STRIPE, DD), f32)
    W_hd = jnp.concatenate(
        [W_pred, W_diff, jnp.zeros((DD, DD - 2), f32)], axis=1)
    b_hd = jnp.concatenate(
        [b_pred, b_diff, jnp.zeros((DD - 2,), f32)]).reshape(1, DD)

    acc1, deg = _segsum_deg(x, packed, zeros_d)
    h, inv = _tc_layer1(acc1.reshape(NC, N_PAD // DD, DD, DD), deg, x_pad,
                        W1_l, W1_r, b1.reshape(1, DD))
    acc2 = _segsum_plain(h, packed, zeros_d)
    o1, o2 = _tc_layer2(acc2.reshape(NC, N_PAD // DD, DD, DD), h, inv,
                        W2_l, W2_r, b2.reshape(1, DD), W_hd, b_hd)
    return (o1[:NN, 0:1], o2[:NN, 0:1])


# speedup vs baseline: 1.1988x; 1.1988x over previous
import dataclasses
import functools

import jax
import jax.numpy as jnp
from jax import lax
from jax.experimental import pallas as pl
from jax.experimental.pallas import tpu as pltpu
from jax.experimental.pallas import tpu_sc as plsc

NN = 10000
N_PAD = 10240
EE = 320000
DD = 128
NC = 2
NS = 16
CHUNK = 128
WIN = 40
FAST_WIN = 3
SLOW_WIN = 1
FAST_CORE = 0
PACK_ROWS = 80
TOT_CHUNKS = NS * WIN * (FAST_WIN + SLOW_WIN)
E_PAD = TOT_CHUNKS * CHUNK
STRIPE = N_PAD // NS
ROW_BLK = 1024
N_BLKS = N_PAD // ROW_BLK


def _sc_params():
    cp = pltpu.CompilerParams(use_tc_tiling_on_sc=False)
    if "needs_layout_passes" in pltpu.CompilerParams.__dataclass_fields__:
        cp = dataclasses.replace(cp, needs_layout_passes=False)
    return cp


def _make_segsum(with_deg):
    mesh = plsc.VectorSubcoreMesh(core_axis_name="c", subcore_axis_name="s")
    out_type = [jax.ShapeDtypeStruct((NC, N_PAD, DD), jnp.float32)]
    scratch = [
        pltpu.VMEM_SHARED((N_PAD, DD), jnp.float32),
        pltpu.VMEM((WIN, CHUNK), jnp.int32),
        pltpu.VMEM((CHUNK,), jnp.int32),
        pltpu.VMEM((CHUNK,), jnp.int32),
        pltpu.VMEM((CHUNK,), jnp.int32),
        pltpu.VMEM((CHUNK,), jnp.int32),
        pltpu.VMEM((CHUNK, DD), jnp.float32),
        pltpu.VMEM((CHUNK, DD), jnp.float32),
    ]
    scratch += [pltpu.SemaphoreType.DMA] * 4
    if with_deg:
        out_type.append(
            jax.ShapeDtypeStruct((NC * NS, N_PAD // DD, DD), jnp.float32))
        scratch.insert(6, pltpu.VMEM((N_PAD // DD, DD), jnp.float32))

    @functools.partial(
        pl.kernel,
        mesh=mesh,
        compiler_params=_sc_params(),
        out_type=tuple(out_type) if with_deg else out_type[0],
        scratch_types=scratch,
    )
    def seg(table_hbm, packed_hbm, zeros_hbm, *refs):
        refs = list(refs)
        out_hbm = refs.pop(0)
        deg_hbm = refs.pop(0) if with_deg else None
        acc_sh = refs.pop(0)
        packed = refs.pop(0)
        idx_s = [refs.pop(0), refs.pop(0)]
        idx_d = [refs.pop(0), refs.pop(0)]
        hist = refs.pop(0) if (with_deg) else None
        rows = [refs.pop(0), refs.pop(0)]
        gsem = [refs.pop(0), refs.pop(0)]
        ssem = [refs.pop(0), refs.pop(0)]

        c = lax.axis_index("c")
        s = lax.axis_index("s")
        wid = c * NS + s
        pltpu.sync_copy(zeros_hbm.at[pl.ds(0, STRIPE)],
                        acc_sh.at[pl.ds(s * STRIPE, STRIPE)])
        if with_deg:
            pltpu.sync_copy(zeros_hbm.at[pl.ds(0, N_PAD // DD)], hist)
        plsc.subcore_barrier()
        fast = c == FAST_CORE
        n_win = jnp.where(fast, FAST_WIN, SLOW_WIN)
        win0 = jnp.where(fast, s * FAST_WIN, NS * FAST_WIN + s * SLOW_WIN)

        def unpack(k, b):
            for c0 in range(0, CHUNK, 16):
                w = packed[k, pl.ds(c0, 16)]
                d = lax.shift_right_logical(w, 16)
                idx_s[b][pl.ds(c0, 16)] = lax.bitwise_and(w, 0xFFFF)
                idx_d[b][pl.ds(c0, 16)] = d
                if with_deg:
                    plsc.addupdate_scatter(
                        hist,
                        [lax.shift_right_logical(d, 7),
                         lax.bitwise_and(d, 0x7F)],
                        jnp.full((16,), 1.0, jnp.float32))

        def g_start(b):
            pltpu.async_copy(table_hbm.at[idx_s[b]], rows[b], gsem[b])

        def g_wait(b):
            pltpu.make_async_copy(table_hbm.at[pl.ds(0, CHUNK)], rows[b],
                                  gsem[b]).wait()

        def s_start(b):
            pltpu.async_copy(rows[b], acc_sh.at[idx_d[b]], ssem[b], add=True)

        def s_wait(b):
            pltpu.make_async_copy(table_hbm.at[pl.ds(0, CHUNK)], rows[b],
                                  ssem[b]).wait()

        @pl.loop(0, n_win)
        def _(wdx):
            pltpu.sync_copy(packed_hbm.at[win0 + wdx], packed)
            for b in range(2):
                unpack(b, b)
                g_start(b)

            @pl.loop(0, WIN, step=2)
            def _(j):
                for b in range(2):
                    k = j + b
                    g_wait(b)
                    s_start(b)

                    @pl.when(k + 2 < WIN)
                    def _():
                        s_wait(b)
                        unpack(k + 2, b)
                        g_start(b)

            for b in range(2):
                s_wait(b)

        if with_deg:
            pltpu.sync_copy(hist, deg_hbm.at[wid])
        plsc.subcore_barrier()
        pltpu.sync_copy(acc_sh.at[pl.ds(s * STRIPE, STRIPE)],
                        out_hbm.at[c].at[pl.ds(s * STRIPE, STRIPE)])

    return seg


_segsum_deg = _make_segsum(True)
_segsum_plain = _make_segsum(False)


ROW_SUB = ROW_BLK // DD


def _tc1_body(acc_ref, deg_ref, x_ref, wl_ref, wr_ref, b_ref, h_ref, inv_ref):
    agg = acc_ref[0] + acc_ref[1]
    deg = jnp.sum(deg_ref[...], axis=0)
    inv = 1.0 / jnp.maximum(deg, 1.0)
    m = (agg * inv[:, :, None]).reshape(ROW_BLK, DD)
    h = (jnp.dot(m, wl_ref[...], preferred_element_type=jnp.float32)
         + jnp.dot(x_ref[...], wr_ref[...], preferred_element_type=jnp.float32)
         + b_ref[...])
    h_ref[...] = jnp.maximum(h, 0.0)
    inv_ref[...] = inv


def _tc2_body(acc_ref, h_ref, inv_ref, wl_ref, wr_ref, b_ref, whd_ref, bhd_ref,
              o1_ref, o2_ref):
    ssum = acc_ref[0] + acc_ref[1]
    m = (ssum * inv_ref[...][:, :, None]).reshape(ROW_BLK, DD)
    h2 = (jnp.dot(m, wl_ref[...], preferred_element_type=jnp.float32)
          + jnp.dot(h_ref[...], wr_ref[...], preferred_element_type=jnp.float32)
          + b_ref[...])
    h2 = jnp.maximum(h2, 0.0)
    t = jnp.dot(h2, whd_ref[...], preferred_element_type=jnp.float32) + bhd_ref[...]
    preds = t[:, 0:1]
    diffs = jax.nn.sigmoid(t[:, 1:2])
    o1_ref[...] = jnp.broadcast_to(preds - diffs, (ROW_BLK, DD))
    o2_ref[...] = jnp.broadcast_to(preds + diffs, (ROW_BLK, DD))


def _full(shape):
    return pl.BlockSpec(shape, lambda j: tuple(0 for _ in shape))


def _tc_layer1(acc1, deg, x_pad, W1_l, W1_r, b1):
    return pl.pallas_call(
        _tc1_body,
        grid=(N_BLKS,),
        in_specs=[
            pl.BlockSpec((NC, ROW_SUB, DD, DD), lambda j: (0, j, 0, 0)),
            pl.BlockSpec((NC * NS, ROW_SUB, DD), lambda j: (0, j, 0)),
            pl.BlockSpec((ROW_BLK, DD), lambda j: (j, 0)),
            _full((DD, DD)),
            _full((DD, DD)),
            _full((1, DD)),
        ],
        out_specs=[
            pl.BlockSpec((ROW_BLK, DD), lambda j: (j, 0)),
            pl.BlockSpec((ROW_SUB, DD), lambda j: (j, 0)),
        ],
        out_shape=[
            jax.ShapeDtypeStruct((N_PAD, DD), jnp.float32),
            jax.ShapeDtypeStruct((N_PAD // DD, DD), jnp.float32),
        ],
    )(acc1, deg, x_pad, W1_l, W1_r, b1)


def _tc_layer2(acc2, h, inv, W2_l, W2_r, b2, W_hd, b_hd):
    return pl.pallas_call(
        _tc2_body,
        grid=(N_BLKS,),
        in_specs=[
            pl.BlockSpec((NC, ROW_SUB, DD, DD), lambda j: (0, j, 0, 0)),
            pl.BlockSpec((ROW_BLK, DD), lambda j: (j, 0)),
            pl.BlockSpec((ROW_SUB, DD), lambda j: (j, 0)),
            _full((DD, DD)),
            _full((DD, DD)),
            _full((1, DD)),
            _full((DD, DD)),
            _full((1, DD)),
        ],
        out_specs=[
            pl.BlockSpec((ROW_BLK, DD), lambda j: (j, 0)),
            pl.BlockSpec((ROW_BLK, DD), lambda j: (j, 0)),
        ],
        out_shape=[
            jax.ShapeDtypeStruct((N_PAD, DD), jnp.float32),
            jax.ShapeDtypeStruct((N_PAD, DD), jnp.float32),
        ],
    )(acc2, h, inv, W2_l, W2_r, b2, W_hd, b_hd)


def kernel(x, edge_index, W1_l, W1_r, b1, W2_l, W2_r, b2, W_pred, b_pred,
           W_diff, b_diff):
    f32 = jnp.float32
    packed = jnp.bitwise_or(edge_index[0],
                            jnp.left_shift(edge_index[1], 16))
    packed = jnp.concatenate(
        [packed, jnp.full((E_PAD - EE,), NN << 16, jnp.int32)]).reshape(
            TOT_CHUNKS // WIN, WIN, CHUNK)
    x_pad = jnp.concatenate([x, jnp.zeros((N_PAD - NN, DD), f32)], axis=0)
    zeros_d = jnp.zeros((STRIPE, DD), f32)
    W_hd = jnp.concatenate(
        [W_pred, W_diff, jnp.zeros((DD, DD - 2), f32)], axis=1)
    b_hd = jnp.concatenate(
        [b_pred, b_diff, jnp.zeros((DD - 2,), f32)]).reshape(1, DD)

    acc1, deg = _segsum_deg(x, packed, zeros_d)
    h, inv = _tc_layer1(acc1.reshape(NC, N_PAD // DD, DD, DD), deg, x_pad,
                        W1_l, W1_r, b1.reshape(1, DD))
    acc2 = _segsum_plain(h, packed, zeros_d)
    o1, o2 = _tc_layer2(acc2.reshape(NC, N_PAD // DD, DD, DD), h, inv,
                        W2_l, W2_r, b2.reshape(1, DD), W_hd, b_hd)
    return (o1[:NN, 0:1], o2[:NN, 0:1])


# speedup vs baseline: 1.2495x; 1.0423x over previous
import dataclasses
import functools

import jax
import jax.numpy as jnp
from jax import lax
from jax.experimental import pallas as pl
from jax.experimental.pallas import tpu as pltpu
from jax.experimental.pallas import tpu_sc as plsc

NN = 10000
N_PAD = 10240
EE = 320000
DD = 128
NC = 2
NS = 16
CHUNK = 128
WIN = 40
FAST_WIN = 3
SLOW_WIN = 1
FAST_CORE = 1
PACK_ROWS = 80
TOT_CHUNKS = NS * WIN * (FAST_WIN + SLOW_WIN)
E_PAD = TOT_CHUNKS * CHUNK
STRIPE = N_PAD // NS
ROW_BLK = 1024
N_BLKS = N_PAD // ROW_BLK


def _sc_params():
    cp = pltpu.CompilerParams(use_tc_tiling_on_sc=False)
    if "needs_layout_passes" in pltpu.CompilerParams.__dataclass_fields__:
        cp = dataclasses.replace(cp, needs_layout_passes=False)
    return cp


def _make_segsum(with_deg):
    mesh = plsc.VectorSubcoreMesh(core_axis_name="c", subcore_axis_name="s")
    out_type = [jax.ShapeDtypeStruct((NC, N_PAD, DD), jnp.float32)]
    scratch = [
        pltpu.VMEM_SHARED((N_PAD, DD), jnp.float32),
        pltpu.VMEM((WIN, CHUNK), jnp.int32),
        pltpu.VMEM((CHUNK,), jnp.int32),
        pltpu.VMEM((CHUNK,), jnp.int32),
        pltpu.VMEM((CHUNK,), jnp.int32),
        pltpu.VMEM((CHUNK,), jnp.int32),
        pltpu.VMEM((CHUNK, DD), jnp.float32),
        pltpu.VMEM((CHUNK, DD), jnp.float32),
    ]
    scratch += [pltpu.SemaphoreType.DMA] * 4
    if with_deg:
        out_type.append(
            jax.ShapeDtypeStruct((NC * NS, N_PAD // DD, DD), jnp.float32))
        scratch.insert(6, pltpu.VMEM((N_PAD // DD, DD), jnp.float32))

    @functools.partial(
        pl.kernel,
        mesh=mesh,
        compiler_params=_sc_params(),
        out_type=tuple(out_type) if with_deg else out_type[0],
        scratch_types=scratch,
    )
    def seg(table_hbm, packed_hbm, zeros_hbm, *refs):
        refs = list(refs)
        out_hbm = refs.pop(0)
        deg_hbm = refs.pop(0) if with_deg else None
        acc_sh = refs.pop(0)
        packed = refs.pop(0)
        idx_s = [refs.pop(0), refs.pop(0)]
        idx_d = [refs.pop(0), refs.pop(0)]
        hist = refs.pop(0) if (with_deg) else None
        rows = [refs.pop(0), refs.pop(0)]
        gsem = [refs.pop(0), refs.pop(0)]
        ssem = [refs.pop(0), refs.pop(0)]

        c = lax.axis_index("c")
        s = lax.axis_index("s")
        wid = c * NS + s
        pltpu.sync_copy(zeros_hbm.at[pl.ds(0, STRIPE)],
                        acc_sh.at[pl.ds(s * STRIPE, STRIPE)])
        if with_deg:
            pltpu.sync_copy(zeros_hbm.at[pl.ds(0, N_PAD // DD)], hist)
        plsc.subcore_barrier()
        fast = c == FAST_CORE
        n_win = jnp.where(fast, FAST_WIN, SLOW_WIN)
        win0 = jnp.where(fast, s * FAST_WIN, NS * FAST_WIN + s * SLOW_WIN)

        def unpack(k, b):
            for c0 in range(0, CHUNK, 16):
                w = packed[k, pl.ds(c0, 16)]
                d = lax.shift_right_logical(w, 16)
                idx_s[b][pl.ds(c0, 16)] = lax.bitwise_and(w, 0xFFFF)
                idx_d[b][pl.ds(c0, 16)] = d
                if with_deg:
                    plsc.addupdate_scatter(
                        hist,
                        [lax.shift_right_logical(d, 7),
                         lax.bitwise_and(d, 0x7F)],
                        jnp.full((16,), 1.0, jnp.float32))

        def g_start(b):
            pltpu.async_copy(table_hbm.at[idx_s[b]], rows[b], gsem[b])

        def g_wait(b):
            pltpu.make_async_copy(table_hbm.at[pl.ds(0, CHUNK)], rows[b],
                                  gsem[b]).wait()

        def s_start(b):
            pltpu.async_copy(rows[b], acc_sh.at[idx_d[b]], ssem[b], add=True)

        def s_wait(b):
            pltpu.make_async_copy(table_hbm.at[pl.ds(0, CHUNK)], rows[b],
                                  ssem[b]).wait()

        @pl.loop(0, n_win)
        def _(wdx):
            pltpu.sync_copy(packed_hbm.at[win0 + wdx], packed)
            for b in range(2):
                unpack(b, b)
                g_start(b)

            @pl.loop(0, WIN, step=2)
            def _(j):
                for b in range(2):
                    k = j + b
                    g_wait(b)
                    s_start(b)

                    @pl.when(k + 2 < WIN)
                    def _():
                        s_wait(b)
                        unpack(k + 2, b)
                        g_start(b)

            for b in range(2):
                s_wait(b)

        if with_deg:
            pltpu.sync_copy(hist, deg_hbm.at[wid])
        plsc.subcore_barrier()
        pltpu.sync_copy(acc_sh.at[pl.ds(s * STRIPE, STRIPE)],
                        out_hbm.at[c].at[pl.ds(s * STRIPE, STRIPE)])

    return seg


_segsum_deg = _make_segsum(True)
_segsum_plain = _make_segsum(False)


ROW_SUB = ROW_BLK // DD


def _tc1_body(acc_ref, deg_ref, x_ref, wl_ref, wr_ref, b_ref, h_ref, inv_ref):
    agg = acc_ref[0] + acc_ref[1]
    deg = jnp.sum(deg_ref[...], axis=0)
    inv = 1.0 / jnp.maximum(deg, 1.0)
    m = (agg * inv[:, :, None]).reshape(ROW_BLK, DD)
    h = (jnp.dot(m, wl_ref[...], preferred_element_type=jnp.float32)
         + jnp.dot(x_ref[...], wr_ref[...], preferred_element_type=jnp.float32)
         + b_ref[...])
    h_ref[...] = jnp.maximum(h, 0.0)
    inv_ref[...] = inv


def _tc2_body(acc_ref, h_ref, inv_ref, wl_ref, wr_ref, b_ref, whd_ref, bhd_ref,
              o1_ref, o2_ref):
    ssum = acc_ref[0] + acc_ref[1]
    m = (ssum * inv_ref[...][:, :, None]).reshape(ROW_BLK, DD)
    h2 = (jnp.dot(m, wl_ref[...], preferred_element_type=jnp.float32)
          + jnp.dot(h_ref[...], wr_ref[...], preferred_element_type=jnp.float32)
          + b_ref[...])
    h2 = jnp.maximum(h2, 0.0)
    t = jnp.dot(h2, whd_ref[...], preferred_element_type=jnp.float32) + bhd_ref[...]
    preds = t[:, 0:1]
    diffs = jax.nn.sigmoid(t[:, 1:2])
    o1_ref[...] = jnp.broadcast_to(preds - diffs, (ROW_BLK, DD))
    o2_ref[...] = jnp.broadcast_to(preds + diffs, (ROW_BLK, DD))


def _full(shape):
    return pl.BlockSpec(shape, lambda j: tuple(0 for _ in shape))


def _tc_layer1(acc1, deg, x_pad, W1_l, W1_r, b1):
    return pl.pallas_call(
        _tc1_body,
        grid=(N_BLKS,),
        in_specs=[
            pl.BlockSpec((NC, ROW_SUB, DD, DD), lambda j: (0, j, 0, 0)),
            pl.BlockSpec((NC * NS, ROW_SUB, DD), lambda j: (0, j, 0)),
            pl.BlockSpec((ROW_BLK, DD), lambda j: (j, 0)),
            _full((DD, DD)),
            _full((DD, DD)),
            _full((1, DD)),
        ],
        out_specs=[
            pl.BlockSpec((ROW_BLK, DD), lambda j: (j, 0)),
            pl.BlockSpec((ROW_SUB, DD), lambda j: (j, 0)),
        ],
        out_shape=[
            jax.ShapeDtypeStruct((N_PAD, DD), jnp.float32),
            jax.ShapeDtypeStruct((N_PAD // DD, DD), jnp.float32),
        ],
    )(acc1, deg, x_pad, W1_l, W1_r, b1)


def _tc_layer2(acc2, h, inv, W2_l, W2_r, b2, W_hd, b_hd):
    return pl.pallas_call(
        _tc2_body,
        grid=(N_BLKS,),
        in_specs=[
            pl.BlockSpec((NC, ROW_SUB, DD, DD), lambda j: (0, j, 0, 0)),
            pl.BlockSpec((ROW_BLK, DD), lambda j: (j, 0)),
            pl.BlockSpec((ROW_SUB, DD), lambda j: (j, 0)),
            _full((DD, DD)),
            _full((DD, DD)),
            _full((1, DD)),
            _full((DD, DD)),
            _full((1, DD)),
        ],
        out_specs=[
            pl.BlockSpec((ROW_BLK, DD), lambda j: (j, 0)),
            pl.BlockSpec((ROW_BLK, DD), lambda j: (j, 0)),
        ],
        out_shape=[
            jax.ShapeDtypeStruct((N_PAD, DD), jnp.float32),
            jax.ShapeDtypeStruct((N_PAD, DD), jnp.float32),
        ],
    )(acc2, h, inv, W2_l, W2_r, b2, W_hd, b_hd)


def kernel(x, edge_index, W1_l, W1_r, b1, W2_l, W2_r, b2, W_pred, b_pred,
           W_diff, b_diff):
    f32 = jnp.float32
    packed = jnp.bitwise_or(edge_index[0],
                            jnp.left_shift(edge_index[1], 16))
    packed = jnp.concatenate(
        [packed, jnp.full((E_PAD - EE,), NN << 16, jnp.int32)]).reshape(
            TOT_CHUNKS // WIN, WIN, CHUNK)
    x_pad = jnp.concatenate([x, jnp.zeros((N_PAD - NN, DD), f32)], axis=0)
    zeros_d = jnp.zeros((STRIPE, DD), f32)
    W_hd = jnp.concatenate(
        [W_pred, W_diff, jnp.zeros((DD, DD - 2), f32)], axis=1)
    b_hd = jnp.concatenate(
        [b_pred, b_diff, jnp.zeros((DD - 2,), f32)]).reshape(1, DD)

    acc1, deg = _segsum_deg(x, packed, zeros_d)
    h, inv = _tc_layer1(acc1.reshape(NC, N_PAD // DD, DD, DD), deg, x_pad,
                        W1_l, W1_r, b1.reshape(1, DD))
    acc2 = _segsum_plain(h, packed, zeros_d)
    o1, o2 = _tc_layer2(acc2.reshape(NC, N_PAD // DD, DD, DD), h, inv,
                        W2_l, W2_r, b2.reshape(1, DD), W_hd, b_hd)
    return (o1[:NN, 0:1], o2[:NN, 0:1])
